# baseline (speedup 1.0000x reference)
import functools

import jax
import jax.numpy as jnp
from jax import lax
from jax.experimental import pallas as pl
from jax.experimental.pallas import tpu as pltpu
from jax.experimental.pallas import tpu_sc as plsc

_NC, _NS = 2, 16
_NW = _NC * _NS


def _sc_add(B, S, D, R, U=4):
    mesh = plsc.VectorSubcoreMesh(core_axis_name="c", subcore_axis_name="s")
    rpw = S // _NW
    nchunks = rpw // R
    G = R * D // 16
    RD = R * D

    @functools.partial(
        pl.kernel,
        mesh=mesh,
        out_type=jax.ShapeDtypeStruct((B * S * D,), jnp.float32),
        scratch_types=(
            [pltpu.VMEM((RD,), jnp.float32) for _ in range(2 * (B + 1))]
            + [pltpu.SemaphoreType.DMA] * 4
        ),
    )
    def k(x_hbm, emb_hbm, out_hbm, *sc):
        emb_v = (sc[0], sc[1])
        x_v = (sc[2:2 + B], sc[2 + B:2 + 2 * B])
        sem_in = (sc[-4], sc[-3])
        sem_out = (sc[-2], sc[-1])
        cid = lax.axis_index("c")
        sid = lax.axis_index("s")
        wid = sid * _NC + cid
        row0 = wid * rpw

        def in_cps(c, s):
            row = row0 + c * R
            cps = [pltpu.make_async_copy(
                emb_hbm.at[pl.ds(row * D, RD)], emb_v[s], sem_in[s])]
            for b in range(B):
                cps.append(pltpu.make_async_copy(
                    x_hbm.at[pl.ds((b * S + row) * D, RD)], x_v[s][b], sem_in[s]))
            return cps

        def out_cps(c, s):
            row = row0 + c * R
            return [pltpu.make_async_copy(
                x_v[s][b], out_hbm.at[pl.ds((b * S + row) * D, RD)], sem_out[s])
                for b in range(B)]

        for cp in in_cps(0, 0):
            cp.start()

        def outer(c2, carry):
            for s in (0, 1):
                c = 2 * c2 + s
                so = 1 - s

                @pl.when(c >= 1)
                def _():
                    for cp in out_cps(c - 1, so):
                        cp.wait()

                @pl.when(c + 1 < nchunks)
                def _():
                    for cp in in_cps(c + 1, so):
                        cp.start()

                for cp in in_cps(c, s):
                    cp.wait()

                @plsc.parallel_loop(0, G, unroll=U)
                def _(g):
                    o = g * 16
                    e = emb_v[s][pl.ds(o, 16)]
                    for b in range(B):
                        x_v[s][b][pl.ds(o, 16)] = x_v[s][b][pl.ds(o, 16)] + e
                for cp in out_cps(c, s):
                    cp.start()
            return carry

        lax.fori_loop(0, nchunks // 2, outer, 0)
        for cp in out_cps(nchunks - 1, (nchunks - 1) % 2):
            cp.wait()

    return k


def _tc_body(x_ref, emb_ref, o_ref):
    o_ref[...] = x_ref[...] + emb_ref[...]


def _tc_add(x, emb):
    B, S, D = x.shape
    BS = 512
    return pl.pallas_call(
        _tc_body,
        grid=(S // BS,),
        in_specs=[
            pl.BlockSpec((B, BS, D), lambda s: (0, s, 0)),
            pl.BlockSpec((BS, D), lambda s: (s, 0)),
        ],
        out_specs=pl.BlockSpec((B, BS, D), lambda s: (0, s, 0)),
        out_shape=jax.ShapeDtypeStruct(x.shape, x.dtype),
    )(x, emb)


def kernel(x, emb):
    B, S, D = x.shape
    xf = x.reshape(B * S * D)
    ef = emb.reshape(-1)[: S * D]
    out = _sc_add(B, S, D, 8)(xf, ef)
    return out.reshape(B, S, D)


# speedup vs baseline: 1.1177x; 1.1177x over previous
import functools

import jax
import jax.numpy as jnp
from jax import lax
from jax.experimental import pallas as pl
from jax.experimental.pallas import tpu as pltpu
from jax.experimental.pallas import tpu_sc as plsc

_NC, _NS = 2, 16
_NW = _NC * _NS


def _sc_add(B, S, D, R, U=4):
    mesh = plsc.VectorSubcoreMesh(core_axis_name="c", subcore_axis_name="s")
    rpw = S // _NW
    nchunks = rpw // R
    G = R * D // 16
    RD = R * D

    @functools.partial(
        pl.kernel,
        mesh=mesh,
        out_type=jax.ShapeDtypeStruct((B, S * D), jnp.float32),
        scratch_types=(
            [pltpu.VMEM((RD,), jnp.float32) for _ in range(2)]
            + [pltpu.VMEM((B, RD), jnp.float32) for _ in range(2)]
            + [pltpu.SemaphoreType.DMA] * 4
        ),
    )
    def k(x_hbm, emb_hbm, out_hbm, e0, e1, xv0, xv1, si0, si1, so0, so1):
        emb_v = (e0, e1)
        x_v = (xv0, xv1)
        sem_in = (si0, si1)
        sem_out = (so0, so1)
        cid = lax.axis_index("c")
        sid = lax.axis_index("s")
        wid = sid * _NC + cid
        row0 = wid * rpw

        def in_cps(c, s):
            row = row0 + c * R
            return [
                pltpu.make_async_copy(
                    emb_hbm.at[pl.ds(row * D, RD)], emb_v[s], sem_in[s]),
                pltpu.make_async_copy(
                    x_hbm.at[:, pl.ds(row * D, RD)], x_v[s], sem_in[s]),
            ]

        def out_cps(c, s):
            row = row0 + c * R
            return [pltpu.make_async_copy(
                x_v[s], out_hbm.at[:, pl.ds(row * D, RD)], sem_out[s])]

        for cp in in_cps(0, 0):
            cp.start()

        def outer(c2, carry):
            for s in (0, 1):
                c = 2 * c2 + s
                so = 1 - s

                @pl.when(c >= 1)
                def _():
                    for cp in out_cps(c - 1, so):
                        cp.wait()

                @pl.when(c + 1 < nchunks)
                def _():
                    for cp in in_cps(c + 1, so):
                        cp.start()

                for cp in in_cps(c, s):
                    cp.wait()

                @plsc.parallel_loop(0, G, unroll=U)
                def _(g):
                    o = g * 16
                    e = emb_v[s][pl.ds(o, 16)]
                    for b in range(B):
                        x_v[s][b, pl.ds(o, 16)] = x_v[s][b, pl.ds(o, 16)] + e

                for cp in out_cps(c, s):
                    cp.start()
            return carry

        lax.fori_loop(0, nchunks // 2, outer, 0)
        for cp in out_cps(nchunks - 1, (nchunks - 1) % 2):
            cp.wait()

    return k


def _tc_body(x_ref, emb_ref, o_ref):
    o_ref[...] = x_ref[...] + emb_ref[...]


def _tc_add(x, emb):
    B, S, D = x.shape
    BS = 512
    return pl.pallas_call(
        _tc_body,
        grid=(S // BS,),
        in_specs=[
            pl.BlockSpec((B, BS, D), lambda s: (0, s, 0)),
            pl.BlockSpec((BS, D), lambda s: (s, 0)),
        ],
        out_specs=pl.BlockSpec((B, BS, D), lambda s: (0, s, 0)),
        out_shape=jax.ShapeDtypeStruct(x.shape, x.dtype),
    )(x, emb)


def kernel(x, emb):
    B, S, D = x.shape
    xf = x.reshape(B, S * D)
    ef = emb.reshape(-1)[: S * D]
    out = _sc_add(B, S, D, 8)(xf, ef)
    return out.reshape(B, S, D)


# speedup vs baseline: 1.5260x; 1.3653x over previous
import functools

import jax
import jax.numpy as jnp
from jax import lax
from jax.experimental import pallas as pl
from jax.experimental.pallas import tpu as pltpu
from jax.experimental.pallas import tpu_sc as plsc

_NC, _NS = 2, 16
_NW = _NC * _NS


def _sc_add(B, S, D, R, s0, s_sc, U=4):
    mesh = plsc.VectorSubcoreMesh(core_axis_name="c", subcore_axis_name="s")
    rpw = s_sc // _NW
    nchunks = rpw // R
    G = R * D // 16
    RD = R * D

    @functools.partial(
        pl.kernel,
        mesh=mesh,
        out_type=jax.ShapeDtypeStruct((B, s_sc * D), jnp.float32),
        scratch_types=(
            [pltpu.VMEM((RD,), jnp.float32) for _ in range(2)]
            + [pltpu.VMEM((B, RD), jnp.float32) for _ in range(2)]
            + [pltpu.SemaphoreType.DMA] * 4
        ),
    )
    def k(x_hbm, emb_hbm, out_hbm, e0, e1, xv0, xv1, si0, si1, so0, so1):
        emb_v = (e0, e1)
        x_v = (xv0, xv1)
        sem_in = (si0, si1)
        sem_out = (so0, so1)
        cid = lax.axis_index("c")
        sid = lax.axis_index("s")
        wid = sid * _NC + cid
        row0 = s0 + wid * rpw

        def in_cps(c, s):
            row = row0 + c * R
            return [
                pltpu.make_async_copy(
                    emb_hbm.at[pl.ds(row * D, RD)], emb_v[s], sem_in[s]),
                pltpu.make_async_copy(
                    x_hbm.at[:, pl.ds(row * D, RD)], x_v[s], sem_in[s]),
            ]

        def out_cps(c, s):
            row = row0 + c * R
            return [pltpu.make_async_copy(
                x_v[s], out_hbm.at[:, pl.ds((row - s0) * D, RD)], sem_out[s])]

        for cp in in_cps(0, 0):
            cp.start()

        def outer(c2, carry):
            for s in (0, 1):
                c = 2 * c2 + s
                so = 1 - s

                @pl.when(c >= 1)
                def _():
                    for cp in out_cps(c - 1, so):
                        cp.wait()

                @pl.when(c + 1 < nchunks)
                def _():
                    for cp in in_cps(c + 1, so):
                        cp.start()

                for cp in in_cps(c, s):
                    cp.wait()

                @plsc.parallel_loop(0, G, unroll=U)
                def _(g):
                    o = g * 16
                    e = emb_v[s][pl.ds(o, 16)]
                    for b in range(B):
                        x_v[s][b, pl.ds(o, 16)] = x_v[s][b, pl.ds(o, 16)] + e

                for cp in out_cps(c, s):
                    cp.start()
            return carry

        lax.fori_loop(0, nchunks // 2, outer, 0)
        for cp in out_cps(nchunks - 1, (nchunks - 1) % 2):
            cp.wait()

    return k


def _tc_body(x_ref, emb_ref, o_ref):
    o_ref[...] = x_ref[...] + emb_ref[...]


def _tc_add(x, emb, S_tc, BS=512):
    B, S, D = x.shape
    return pl.pallas_call(
        _tc_body,
        grid=(S_tc // BS,),
        in_specs=[
            pl.BlockSpec((B, BS, D), lambda s: (0, s, 0)),
            pl.BlockSpec((BS, D), lambda s: (s, 0)),
        ],
        out_specs=pl.BlockSpec((B, BS, D), lambda s: (0, s, 0)),
        out_shape=jax.ShapeDtypeStruct((B, S, D), x.dtype),
    )(x, emb)


_S_SC = 1024


def kernel(x, emb):
    B, S, D = x.shape
    S_tc = S - _S_SC
    xf = x.reshape(B, S * D)
    ef = emb.reshape(-1)[: S * D]
    out_sc = _sc_add(B, S, D, 8, S_tc, _S_SC)(xf, ef)
    out_tc = _tc_add(x, emb, S_tc)
    return lax.dynamic_update_slice(
        out_tc, out_sc.reshape(B, _S_SC, D), (0, S_tc, 0))


# speedup vs baseline: 4.1410x; 2.7136x over previous
import functools

import jax
import jax.numpy as jnp
from jax import lax
from jax.experimental import pallas as pl
from jax.experimental.pallas import tpu as pltpu
from jax.experimental.pallas import tpu_sc as plsc

_NC, _NS = 2, 16
_NW = _NC * _NS


def _sc_add(B, S, D, R, s0, s_sc, U=4):
    mesh = plsc.VectorSubcoreMesh(core_axis_name="c", subcore_axis_name="s")
    rpw = s_sc // _NW
    nchunks = rpw // R
    G = R * D // 16
    RD = R * D

    @functools.partial(
        pl.kernel,
        mesh=mesh,
        out_type=jax.ShapeDtypeStruct((B, s_sc * D), jnp.float32),
        scratch_types=(
            [pltpu.VMEM((RD,), jnp.float32) for _ in range(2)]
            + [pltpu.VMEM((B, RD), jnp.float32) for _ in range(2)]
            + [pltpu.SemaphoreType.DMA] * 4
        ),
    )
    def k(x_hbm, emb_hbm, out_hbm, e0, e1, xv0, xv1, si0, si1, so0, so1):
        emb_v = (e0, e1)
        x_v = (xv0, xv1)
        sem_in = (si0, si1)
        sem_out = (so0, so1)
        cid = lax.axis_index("c")
        sid = lax.axis_index("s")
        wid = sid * _NC + cid
        row0 = s0 + wid * rpw

        def in_cps(c, s):
            row = row0 + c * R
            return [
                pltpu.make_async_copy(
                    emb_hbm.at[pl.ds(row * D, RD)], emb_v[s], sem_in[s]),
                pltpu.make_async_copy(
                    x_hbm.at[:, pl.ds(row * D, RD)], x_v[s], sem_in[s]),
            ]

        def out_cps(c, s):
            row = row0 + c * R
            return [pltpu.make_async_copy(
                x_v[s], out_hbm.at[:, pl.ds((row - s0) * D, RD)], sem_out[s])]

        for cp in in_cps(0, 0):
            cp.start()

        def outer(c2, carry):
            for s in (0, 1):
                c = 2 * c2 + s
                so = 1 - s

                @pl.when(c >= 1)
                def _():
                    for cp in out_cps(c - 1, so):
                        cp.wait()

                @pl.when(c + 1 < nchunks)
                def _():
                    for cp in in_cps(c + 1, so):
                        cp.start()

                for cp in in_cps(c, s):
                    cp.wait()

                @plsc.parallel_loop(0, G, unroll=U)
                def _(g):
                    o = g * 16
                    e = emb_v[s][pl.ds(o, 16)]
                    for b in range(B):
                        x_v[s][b, pl.ds(o, 16)] = x_v[s][b, pl.ds(o, 16)] + e

                for cp in out_cps(c, s):
                    cp.start()
            return carry

        lax.fori_loop(0, nchunks // 2, outer, 0)
        for cp in out_cps(nchunks - 1, (nchunks - 1) % 2):
            cp.wait()

    return k


def _tc_body(x_ref, emb_ref, o_ref):
    o_ref[...] = x_ref[...] + emb_ref[...]


def _tc_add(x, emb, S_tc, BS=512):
    B, S, D = x.shape
    return pl.pallas_call(
        _tc_body,
        grid=(S_tc // BS,),
        in_specs=[
            pl.BlockSpec((B, BS, D), lambda s: (0, s, 0)),
            pl.BlockSpec((BS, D), lambda s: (s, 0)),
        ],
        out_specs=pl.BlockSpec((B, BS, D), lambda s: (0, s, 0)),
        out_shape=jax.ShapeDtypeStruct((B, S, D), x.dtype),
    )(x, emb)


def kernel(x, emb):
    B, S, D = x.shape
    BS = 1024
    BB = 2
    return pl.pallas_call(
        _tc_body,
        grid=(S // BS, B // BB),
        in_specs=[
            pl.BlockSpec((BB, BS, D), lambda s, b: (b, s, 0)),
            pl.BlockSpec((BS, D), lambda s, b: (s, 0)),
        ],
        out_specs=pl.BlockSpec((BB, BS, D), lambda s, b: (b, s, 0)),
        out_shape=jax.ShapeDtypeStruct(x.shape, x.dtype),
    )(x, emb)


# speedup vs baseline: 4.1436x; 1.0006x over previous
import functools

import jax
import jax.numpy as jnp
from jax import lax
from jax.experimental import pallas as pl
from jax.experimental.pallas import tpu as pltpu
from jax.experimental.pallas import tpu_sc as plsc

_NC, _NS = 2, 16
_NW = _NC * _NS


def _sc_add(B, S, D, R, s0, s_sc, U=4):
    mesh = plsc.VectorSubcoreMesh(core_axis_name="c", subcore_axis_name="s")
    rpw = s_sc // _NW
    nchunks = rpw // R
    G = R * D // 16
    RD = R * D

    @functools.partial(
        pl.kernel,
        mesh=mesh,
        out_type=jax.ShapeDtypeStruct((B, s_sc * D), jnp.float32),
        scratch_types=(
            [pltpu.VMEM((RD,), jnp.float32) for _ in range(2)]
            + [pltpu.VMEM((B, RD), jnp.float32) for _ in range(2)]
            + [pltpu.SemaphoreType.DMA] * 4
        ),
    )
    def k(x_hbm, emb_hbm, out_hbm, e0, e1, xv0, xv1, si0, si1, so0, so1):
        emb_v = (e0, e1)
        x_v = (xv0, xv1)
        sem_in = (si0, si1)
        sem_out = (so0, so1)
        cid = lax.axis_index("c")
        sid = lax.axis_index("s")
        wid = sid * _NC + cid
        row0 = s0 + wid * rpw

        def in_cps(c, s):
            row = row0 + c * R
            return [
                pltpu.make_async_copy(
                    emb_hbm.at[pl.ds(row * D, RD)], emb_v[s], sem_in[s]),
                pltpu.make_async_copy(
                    x_hbm.at[:, pl.ds(row * D, RD)], x_v[s], sem_in[s]),
            ]

        def out_cps(c, s):
            row = row0 + c * R
            return [pltpu.make_async_copy(
                x_v[s], out_hbm.at[:, pl.ds((row - s0) * D, RD)], sem_out[s])]

        for cp in in_cps(0, 0):
            cp.start()

        def outer(c2, carry):
            for s in (0, 1):
                c = 2 * c2 + s
                so = 1 - s

                @pl.when(c >= 1)
                def _():
                    for cp in out_cps(c - 1, so):
                        cp.wait()

                @pl.when(c + 1 < nchunks)
                def _():
                    for cp in in_cps(c + 1, so):
                        cp.start()

                for cp in in_cps(c, s):
                    cp.wait()

                @plsc.parallel_loop(0, G, unroll=U)
                def _(g):
                    o = g * 16
                    e = emb_v[s][pl.ds(o, 16)]
                    for b in range(B):
                        x_v[s][b, pl.ds(o, 16)] = x_v[s][b, pl.ds(o, 16)] + e

                for cp in out_cps(c, s):
                    cp.start()
            return carry

        lax.fori_loop(0, nchunks // 2, outer, 0)
        for cp in out_cps(nchunks - 1, (nchunks - 1) % 2):
            cp.wait()

    return k


def _tc_body(x_ref, emb_ref, o_ref):
    o_ref[...] = x_ref[...] + emb_ref[...]


def _tc_add(x, emb, S_tc, BS=512):
    B, S, D = x.shape
    return pl.pallas_call(
        _tc_body,
        grid=(S_tc // BS,),
        in_specs=[
            pl.BlockSpec((B, BS, D), lambda s: (0, s, 0)),
            pl.BlockSpec((BS, D), lambda s: (s, 0)),
        ],
        out_specs=pl.BlockSpec((B, BS, D), lambda s: (0, s, 0)),
        out_shape=jax.ShapeDtypeStruct((B, S, D), x.dtype),
    )(x, emb)


def kernel(x, emb):
    B, S, D = x.shape
    BS = 2048
    BB = 1
    return pl.pallas_call(
        _tc_body,
        grid=(S // BS, B // BB),
        in_specs=[
            pl.BlockSpec((BB, BS, D), lambda s, b: (b, s, 0)),
            pl.BlockSpec((BS, D), lambda s, b: (s, 0)),
        ],
        out_specs=pl.BlockSpec((BB, BS, D), lambda s, b: (b, s, 0)),
        out_shape=jax.ShapeDtypeStruct(x.shape, x.dtype),
    )(x, emb)


# speedup vs baseline: 4.1532x; 1.0023x over previous
import functools

import jax
import jax.numpy as jnp
from jax import lax
from jax.experimental import pallas as pl
from jax.experimental.pallas import tpu as pltpu
from jax.experimental.pallas import tpu_sc as plsc

_NC, _NS = 2, 16
_NW = _NC * _NS


def _sc_add(B, S, D, R, s0, s_sc, U=4):
    mesh = plsc.VectorSubcoreMesh(core_axis_name="c", subcore_axis_name="s")
    rpw = s_sc // _NW
    nchunks = rpw // R
    G = R * D // 16
    RD = R * D

    @functools.partial(
        pl.kernel,
        mesh=mesh,
        out_type=jax.ShapeDtypeStruct((B, s_sc * D), jnp.float32),
        scratch_types=(
            [pltpu.VMEM((RD,), jnp.float32) for _ in range(2)]
            + [pltpu.VMEM((B, RD), jnp.float32) for _ in range(2)]
            + [pltpu.SemaphoreType.DMA] * 4
        ),
    )
    def k(x_hbm, emb_hbm, out_hbm, e0, e1, xv0, xv1, si0, si1, so0, so1):
        emb_v = (e0, e1)
        x_v = (xv0, xv1)
        sem_in = (si0, si1)
        sem_out = (so0, so1)
        cid = lax.axis_index("c")
        sid = lax.axis_index("s")
        wid = sid * _NC + cid
        row0 = s0 + wid * rpw

        def in_cps(c, s):
            row = row0 + c * R
            return [
                pltpu.make_async_copy(
                    emb_hbm.at[pl.ds(row * D, RD)], emb_v[s], sem_in[s]),
                pltpu.make_async_copy(
                    x_hbm.at[:, pl.ds(row * D, RD)], x_v[s], sem_in[s]),
            ]

        def out_cps(c, s):
            row = row0 + c * R
            return [pltpu.make_async_copy(
                x_v[s], out_hbm.at[:, pl.ds((row - s0) * D, RD)], sem_out[s])]

        for cp in in_cps(0, 0):
            cp.start()

        def outer(c2, carry):
            for s in (0, 1):
                c = 2 * c2 + s
                so = 1 - s

                @pl.when(c >= 1)
                def _():
                    for cp in out_cps(c - 1, so):
                        cp.wait()

                @pl.when(c + 1 < nchunks)
                def _():
                    for cp in in_cps(c + 1, so):
                        cp.start()

                for cp in in_cps(c, s):
                    cp.wait()

                @plsc.parallel_loop(0, G, unroll=U)
                def _(g):
                    o = g * 16
                    e = emb_v[s][pl.ds(o, 16)]
                    for b in range(B):
                        x_v[s][b, pl.ds(o, 16)] = x_v[s][b, pl.ds(o, 16)] + e

                for cp in out_cps(c, s):
                    cp.start()
            return carry

        lax.fori_loop(0, nchunks // 2, outer, 0)
        for cp in out_cps(nchunks - 1, (nchunks - 1) % 2):
            cp.wait()

    return k


def _tc_body(x_ref, emb_ref, o_ref):
    o_ref[...] = x_ref[...] + emb_ref[...]


def kernel(x, emb):
    B, S, D = x.shape
    BS = 2048
    return pl.pallas_call(
        _tc_body,
        grid=(S // BS, B),
        in_specs=[
            pl.BlockSpec((1, BS, D), lambda s, b: (b, s, 0)),
            pl.BlockSpec((BS, D), lambda s, b: (s, 0)),
        ],
        out_specs=pl.BlockSpec((1, BS, D), lambda s, b: (b, s, 0)),
        out_shape=jax.ShapeDtypeStruct(x.shape, x.dtype),
    )(x, emb)
